# baseline (speedup 1.0000x reference)
import functools

import jax
import jax.numpy as jnp
from jax.experimental import pallas as pl
from jax.experimental.pallas import tpu as pltpu


def _body(g_ref, ft_ref, w0_ref, w1_ref, w2_ref, b0_ref, b1_ref, b2_ref,
          out_ref, a16_ref, rc_ref, s_ref, pr_ref, u_ref, acc_ref,
          za_ref, zb_ref, *, bi, ni):
    l = pl.program_id(0)
    i = pl.program_id(1)
    isl = pl.ds(i * bi, bi)

    @pl.when(l == 0)
    def _():
        a = g_ref[...]
        row = jax.lax.broadcasted_iota(jnp.int32, a.shape, 0) + i * bi
        col = jax.lax.broadcasted_iota(jnp.int32, a.shape, 1)
        a = jnp.where(row == col, 1.0, a)
        a16_ref[isl, :] = a.astype(jnp.bfloat16)
        rc_ref[isl, :] = jnp.sum(a, axis=1, keepdims=True)

    @pl.when(l == 1)
    def _():
        p = jax.lax.rsqrt(rc_ref[isl, :])
        s_part = jax.lax.dot_general(
            p, a16_ref[isl, :].astype(jnp.float32),
            (((0,), (0,)), ((), ())),
            preferred_element_type=jnp.float32,
        )
        row = jax.lax.broadcasted_iota(jnp.int32, (bi, s_ref.shape[1]), 0)
        col = jax.lax.broadcasted_iota(jnp.int32, (bi, s_ref.shape[1]), 1)
        ident = (row + i * bi == col).astype(jnp.float32)
        p_part = jax.lax.dot_general(
            p, ident, (((0,), (0,)), ((), ())),
            preferred_element_type=jnp.float32,
        )

        @pl.when(i == 0)
        def _():
            s_ref[...] = s_part
            pr_ref[...] = p_part

        @pl.when(i != 0)
        def _():
            s_ref[...] = s_ref[...] + s_part
            pr_ref[...] = pr_ref[...] + p_part

        @pl.when(i == ni - 1)
        def _():
            u_ref[...] = jax.lax.rsqrt(s_ref[...] / pr_ref[...])

    def layer(zsrc, wt_ref, b_ref, writeback):
        fout = wt_ref.shape[0]
        y = jnp.dot(wt_ref[...], zsrc.astype(jnp.float32),
                    preferred_element_type=jnp.float32)
        y = (y * u_ref[:, isl]).astype(jnp.bfloat16)
        part = jnp.dot(y, a16_ref[isl, :], preferred_element_type=jnp.float32)

        @pl.when(i == 0)
        def _():
            acc_ref[:fout, :] = part

        @pl.when(i != 0)
        def _():
            acc_ref[:fout, :] = acc_ref[:fout, :] + part

        @pl.when(i == ni - 1)
        def _():
            writeback(jnp.maximum(
                acc_ref[:fout, :] * u_ref[...] + b_ref[...], 0.0))

    @pl.when(l == 2)
    def _():
        layer(ft_ref[...], w0_ref, b0_ref,
              lambda v: za_ref.__setitem__((Ellipsis,), v.astype(jnp.bfloat16)))

    @pl.when(l == 3)
    def _():
        layer(za_ref[:, isl], w1_ref, b1_ref,
              lambda v: zb_ref.__setitem__((Ellipsis,), v.astype(jnp.bfloat16)))

    @pl.when(l == 4)
    def _():
        layer(zb_ref[:, isl], w2_ref, b2_ref,
              lambda v: out_ref.__setitem__((Ellipsis,), v))


def kernel(features, graph, W0, b0, W1, b1, W2, b2):
    n = graph.shape[0]
    bi = 128
    ni = n // bi
    d0, h = W0.shape
    latent = W2.shape[1]

    full = lambda shape: pl.BlockSpec(shape, lambda l, i: (0, 0))
    outt = pl.pallas_call(
        functools.partial(_body, bi=bi, ni=ni),
        grid=(5, ni),
        in_specs=[
            pl.BlockSpec((bi, n), lambda l, i: (jnp.where(l == 0, i, 0), 0)),
            pl.BlockSpec((d0, bi), lambda l, i: (0, jnp.where(l == 2, i, 0))),
            full((h, d0)),
            full((h, h)),
            full((latent, h)),
            full((h, 1)),
            full((h, 1)),
            full((latent, 1)),
        ],
        out_specs=full((latent, n)),
        out_shape=jax.ShapeDtypeStruct((latent, n), jnp.float32),
        scratch_shapes=[
            pltpu.VMEM((n, n), jnp.bfloat16),
            pltpu.VMEM((n, 1), jnp.float32),
            pltpu.VMEM((1, n), jnp.float32),
            pltpu.VMEM((1, n), jnp.float32),
            pltpu.VMEM((1, n), jnp.float32),
            pltpu.VMEM((h, n), jnp.float32),
            pltpu.VMEM((h, n), jnp.bfloat16),
            pltpu.VMEM((h, n), jnp.bfloat16),
        ],
        compiler_params=pltpu.CompilerParams(
            dimension_semantics=("arbitrary", "arbitrary")
        ),
    )(graph, features.T, W0.T, W1.T, W2.T,
      b0.reshape(h, 1), b1.reshape(h, 1), b2.reshape(latent, 1))
    return outt.T


# speedup vs baseline: 1.9615x; 1.9615x over previous
import functools

import jax
import jax.numpy as jnp
from jax.experimental import pallas as pl
from jax.experimental.pallas import tpu as pltpu


def _body(g_ref, ft_ref, w0_ref, w1_ref, w2_ref, b0_ref, b1_ref, b2_ref,
          out_ref, a16_ref, rc_ref, s_ref, pr_ref, u_ref, acc_ref,
          za_ref, zb_ref, *, bi, ni):
    l = pl.program_id(0)
    i = pl.program_id(1)
    isl = pl.ds(i * bi, bi)

    @pl.when(l == 0)
    def _():
        a = g_ref[...]
        row = jax.lax.broadcasted_iota(jnp.int32, a.shape, 0) + i * bi
        col = jax.lax.broadcasted_iota(jnp.int32, a.shape, 1)
        a = jnp.where(row == col, 1.0, a)
        a16_ref[isl, :] = a.astype(jnp.int8)
        rc_ref[isl, :] = jnp.sum(a, axis=1, keepdims=True)

    @pl.when(l == 1)
    def _():
        p = jax.lax.rsqrt(rc_ref[isl, :])
        s_part = jax.lax.dot_general(
            p, a16_ref[isl, :].astype(jnp.float32),
            (((0,), (0,)), ((), ())),
            preferred_element_type=jnp.float32,
        )
        row = jax.lax.broadcasted_iota(jnp.int32, (bi, s_ref.shape[1]), 0)
        col = jax.lax.broadcasted_iota(jnp.int32, (bi, s_ref.shape[1]), 1)
        ident = (row + i * bi == col).astype(jnp.float32)
        p_part = jax.lax.dot_general(
            p, ident, (((0,), (0,)), ((), ())),
            preferred_element_type=jnp.float32,
        )

        @pl.when(i == 0)
        def _():
            s_ref[...] = s_part
            pr_ref[...] = p_part

        @pl.when(i != 0)
        def _():
            s_ref[...] = s_ref[...] + s_part
            pr_ref[...] = pr_ref[...] + p_part

        @pl.when(i == ni - 1)
        def _():
            u_ref[...] = jax.lax.rsqrt(s_ref[...] / pr_ref[...])

    def layer(zsrc, wt_ref, b_ref, writeback):
        fout = wt_ref.shape[0]
        y = jnp.dot(wt_ref[...], zsrc.astype(jnp.float32),
                    preferred_element_type=jnp.float32)
        y = (y * u_ref[:, isl]).astype(jnp.bfloat16)
        part = jnp.dot(y, a16_ref[isl, :].astype(jnp.bfloat16),
                       preferred_element_type=jnp.float32)

        @pl.when(i == 0)
        def _():
            acc_ref[:fout, :] = part

        @pl.when(i != 0)
        def _():
            acc_ref[:fout, :] = acc_ref[:fout, :] + part

        @pl.when(i == ni - 1)
        def _():
            writeback(jnp.maximum(
                acc_ref[:fout, :] * u_ref[...] + b_ref[...], 0.0))

    @pl.when(l == 2)
    def _():
        layer(ft_ref[...], w0_ref, b0_ref,
              lambda v: za_ref.__setitem__((Ellipsis,), v.astype(jnp.bfloat16)))

    @pl.when(l == 3)
    def _():
        layer(za_ref[:, isl], w1_ref, b1_ref,
              lambda v: zb_ref.__setitem__((Ellipsis,), v.astype(jnp.bfloat16)))

    @pl.when(l == 4)
    def _():
        layer(zb_ref[:, isl], w2_ref, b2_ref,
              lambda v: out_ref.__setitem__((Ellipsis,), v))


def kernel(features, graph, W0, b0, W1, b1, W2, b2):
    n = graph.shape[0]
    bi = 512
    ni = n // bi
    d0, h = W0.shape
    latent = W2.shape[1]

    full = lambda shape: pl.BlockSpec(shape, lambda l, i: (0, 0))
    outt = pl.pallas_call(
        functools.partial(_body, bi=bi, ni=ni),
        grid=(5, ni),
        in_specs=[
            pl.BlockSpec((bi, n), lambda l, i: (jnp.where(l == 0, i, 0), 0)),
            pl.BlockSpec((d0, bi), lambda l, i: (0, jnp.where(l == 2, i, 0))),
            full((h, d0)),
            full((h, h)),
            full((latent, h)),
            full((h, 1)),
            full((h, 1)),
            full((latent, 1)),
        ],
        out_specs=full((latent, n)),
        out_shape=jax.ShapeDtypeStruct((latent, n), jnp.float32),
        scratch_shapes=[
            pltpu.VMEM((n, n), jnp.int8),
            pltpu.VMEM((n, 1), jnp.float32),
            pltpu.VMEM((1, n), jnp.float32),
            pltpu.VMEM((1, n), jnp.float32),
            pltpu.VMEM((1, n), jnp.float32),
            pltpu.VMEM((h, n), jnp.float32),
            pltpu.VMEM((h, n), jnp.bfloat16),
            pltpu.VMEM((h, n), jnp.bfloat16),
        ],
        compiler_params=pltpu.CompilerParams(
            dimension_semantics=("arbitrary", "arbitrary")
        ),
    )(graph, features.T, W0.T, W1.T, W2.T,
      b0.reshape(h, 1), b1.reshape(h, 1), b2.reshape(latent, 1))
    return outt.T


# speedup vs baseline: 1.9653x; 1.0019x over previous
import functools

import jax
import jax.numpy as jnp
from jax.experimental import pallas as pl
from jax.experimental.pallas import tpu as pltpu


def _body(g_ref, ft_ref, w0_ref, w1_ref, w2_ref, b0_ref, b1_ref, b2_ref,
          out_ref, a16_ref, rc_ref, s_ref, pr_ref, u_ref, acc_ref,
          za_ref, zb_ref, *, bi, ni):
    l = pl.program_id(0)
    i = pl.program_id(1)
    isl = pl.ds(i * bi, bi)

    @pl.when(l == 0)
    def _():
        a = g_ref[...]
        row = jax.lax.broadcasted_iota(jnp.int32, a.shape, 0) + i * bi
        col = jax.lax.broadcasted_iota(jnp.int32, a.shape, 1)
        a = jnp.where(row == col, 1.0, a)
        a16_ref[isl, :] = a.astype(jnp.int8)
        rc_ref[isl, :] = jnp.sum(a, axis=1, keepdims=True)

    @pl.when(l == 1)
    def _():
        p = jax.lax.rsqrt(rc_ref[isl, :])
        s_part = jax.lax.dot_general(
            p, a16_ref[isl, :].astype(jnp.float32),
            (((0,), (0,)), ((), ())),
            preferred_element_type=jnp.float32,
        )
        row = jax.lax.broadcasted_iota(jnp.int32, (bi, s_ref.shape[1]), 0)
        col = jax.lax.broadcasted_iota(jnp.int32, (bi, s_ref.shape[1]), 1)
        ident = (row + i * bi == col).astype(jnp.float32)
        p_part = jax.lax.dot_general(
            p, ident, (((0,), (0,)), ((), ())),
            preferred_element_type=jnp.float32,
        )

        @pl.when(i == 0)
        def _():
            s_ref[...] = s_part
            pr_ref[...] = p_part

        @pl.when(i != 0)
        def _():
            s_ref[...] = s_ref[...] + s_part
            pr_ref[...] = pr_ref[...] + p_part

        @pl.when(i == ni - 1)
        def _():
            u_ref[...] = jax.lax.rsqrt(s_ref[...] / pr_ref[...])

    def layer(zsrc, wt_ref, b_ref, writeback):
        fout = wt_ref.shape[0]
        y = jnp.dot(wt_ref[...], zsrc.astype(jnp.float32),
                    preferred_element_type=jnp.float32)
        y = (y * u_ref[:, isl]).astype(jnp.bfloat16)
        part = jax.lax.dot_general(
            y, a16_ref[isl, :], (((1,), (0,)), ((), ())),
            preferred_element_type=jnp.float32)

        @pl.when(i == 0)
        def _():
            acc_ref[:fout, :] = part

        @pl.when(i != 0)
        def _():
            acc_ref[:fout, :] = acc_ref[:fout, :] + part

        @pl.when(i == ni - 1)
        def _():
            writeback(jnp.maximum(
                acc_ref[:fout, :] * u_ref[...] + b_ref[...], 0.0))

    @pl.when(l == 2)
    def _():
        layer(ft_ref[...], w0_ref, b0_ref,
              lambda v: za_ref.__setitem__((Ellipsis,), v.astype(jnp.bfloat16)))

    @pl.when(l == 3)
    def _():
        layer(za_ref[:, isl], w1_ref, b1_ref,
              lambda v: zb_ref.__setitem__((Ellipsis,), v.astype(jnp.bfloat16)))

    @pl.when(l == 4)
    def _():
        layer(zb_ref[:, isl], w2_ref, b2_ref,
              lambda v: out_ref.__setitem__((Ellipsis,), v))


def kernel(features, graph, W0, b0, W1, b1, W2, b2):
    n = graph.shape[0]
    bi = 512
    ni = n // bi
    d0, h = W0.shape
    latent = W2.shape[1]

    full = lambda shape: pl.BlockSpec(shape, lambda l, i: (0, 0))
    outt = pl.pallas_call(
        functools.partial(_body, bi=bi, ni=ni),
        grid=(5, ni),
        in_specs=[
            pl.BlockSpec((bi, n), lambda l, i: (jnp.where(l == 0, i, 0), 0)),
            pl.BlockSpec((d0, bi), lambda l, i: (0, jnp.where(l == 2, i, 0))),
            full((h, d0)),
            full((h, h)),
            full((latent, h)),
            full((h, 1)),
            full((h, 1)),
            full((latent, 1)),
        ],
        out_specs=full((latent, n)),
        out_shape=jax.ShapeDtypeStruct((latent, n), jnp.float32),
        scratch_shapes=[
            pltpu.VMEM((n, n), jnp.int8),
            pltpu.VMEM((n, 1), jnp.float32),
            pltpu.VMEM((1, n), jnp.float32),
            pltpu.VMEM((1, n), jnp.float32),
            pltpu.VMEM((1, n), jnp.float32),
            pltpu.VMEM((h, n), jnp.float32),
            pltpu.VMEM((h, n), jnp.bfloat16),
            pltpu.VMEM((h, n), jnp.bfloat16),
        ],
        compiler_params=pltpu.CompilerParams(
            dimension_semantics=("arbitrary", "arbitrary")
        ),
    )(graph, features.T, W0.T, W1.T, W2.T,
      b0.reshape(h, 1), b1.reshape(h, 1), b2.reshape(latent, 1))
    return outt.T
